# Initial kernel scaffold; baseline (speedup 1.0000x reference)
import jax
import jax.numpy as jnp
from jax.experimental import pallas as pl


def kernel(z, W1, b1, W2, b2, We, be, gat_W, gat_b, attn_l, attn_r):
    raise NotImplementedError("write your pallas kernel here")


# speedup vs baseline: 529.6952x; 529.6952x over previous
import jax
import jax.numpy as jnp
from jax import lax
from jax.experimental import pallas as pl
from jax.experimental.pallas import tpu as pltpu

_N = 64
_D = 128
_NEG = -1e30

_DN_LAST = (((1,), (1,)), ((), ()))
_DN_S0 = (((0,), (0,)), ((), ()))


def _gen_body(z_ref, w1_ref, w2_ref, b2_ref, we_ref, gatw_ref, gatb_ref,
              al_ref, ar_ref, b1_ref, be_ref, out_ref):
    z = z_ref[...]
    w1 = w1_ref[...]
    w2row = w2_ref[...]
    b2r = b2_ref[...]
    we = we_ref[...]
    gatb = gatb_ref[...]
    al = al_ref[...]
    ar = ar_ref[...]
    b1s = b1_ref[0, 0]
    bes = be_ref[0, 0]

    bias_mean = (gatb[0:1, :] + gatb[1:2, :] + gatb[2:3, :]) * (1.0 / 3.0)

    row_i = lax.broadcasted_iota(jnp.int32, (_N, 1), 0)
    row_f = row_i.astype(jnp.float32)
    d_row_f = lax.broadcasted_iota(jnp.int32, (1, _N), 1).astype(jnp.float32)

    def gat(hh, kcol, n2f):
        mask = d_row_f < kcol
        acc = jnp.zeros((_N, _D), jnp.float32)
        for head in range(3):
            gw = gatw_ref[head * _D:(head + 1) * _D, :]
            feat = lax.dot_general(hh, gw, _DN_LAST,
                                   preferred_element_type=jnp.float32)
            el3 = lax.dot_general(feat, al, _DN_LAST,
                                  preferred_element_type=jnp.float32)
            er3 = lax.dot_general(ar, feat, _DN_LAST,
                                  preferred_element_type=jnp.float32)
            elc = el3[:, head:head + 1]
            err = er3[head:head + 1, :]
            epre = elc + err
            e = jnp.where(epre >= 0, epre, 0.2 * epre)
            em = jnp.where(mask, e, _NEG)
            m = jnp.max(em, axis=0, keepdims=True)
            m = jnp.where(m > 0.1 * _NEG, m, 0.0)
            ex = jnp.where(mask, jnp.exp(e - m), 0.0)
            denom = jnp.sum(ex, axis=0, keepdims=True)
            dsafe = jnp.where(denom > 0, denom, 1.0)
            alpha = ex / dsafe
            acc = acc + lax.dot_general(alpha, feat, _DN_S0,
                                        preferred_element_type=jnp.float32)
        hnew = acc * (1.0 / 3.0) + bias_mean
        hnew = jnp.where(row_f < n2f, hnew, 0.0)
        snew = jnp.sum(hnew, axis=0, keepdims=True) / n2f
        return hnew, snew

    t0pre = jnp.sum(jnp.concatenate([z, z], axis=1) * w1) + b1s
    t0 = jnp.maximum(t0pre, 0.0)
    h0row = t0 * w2row + b2r
    h0 = jnp.where(row_i == 0, h0row, 0.0)
    k0 = jnp.zeros((_N, 1), jnp.float32)
    h0, s0 = gat(h0, k0, jnp.float32(1.0))

    def cond(c):
        return jnp.logical_not(c[4])

    def body(c):
        h, kcol, n, s, _ = c
        tpre = jnp.sum(jnp.concatenate([z, s], axis=1) * w1) + b1s
        stop = jnp.logical_or(tpre <= 0.0, n >= _N)
        tok = jnp.maximum(tpre, 0.0)
        hnrow = tok * w2row + b2r
        h2 = jnp.where(row_i == n, hnrow, h)
        n2 = n + 1
        n2f = n2.astype(jnp.float32)
        big = jnp.concatenate([
            jnp.broadcast_to(z, (_N, _D)),
            jnp.broadcast_to(s, (_N, _D)),
            jnp.broadcast_to(hnrow, (_N, _D)),
            h2,
        ], axis=1)
        te8 = lax.dot_general(big, we, _DN_LAST,
                              preferred_element_type=jnp.float32)
        te = te8[:, 0:1] + bes
        brk = te < 1e-4
        cand = jnp.where(brk, row_f, jnp.float32(_N))
        knew = jnp.minimum(jnp.min(cand), n2f)
        kcol2 = jnp.where(row_i == n, knew, kcol)
        hg, s3 = gat(h2, kcol2, n2f)
        h_o = jnp.where(stop, h, hg)
        k_o = jnp.where(stop, kcol, kcol2)
        n_o = jnp.where(stop, n, n2)
        s_o = jnp.where(stop, s, s3)
        return (h_o, k_o, n_o, s_o, stop)

    final = lax.while_loop(cond, body,
                           (h0, k0, jnp.int32(1), s0, jnp.bool_(False)))
    out_ref[...] = final[0]


def kernel(z, W1, b1, W2, b2, We, be, gat_W, gat_b, attn_l, attn_r):
    f32 = jnp.float32
    vmem = pl.BlockSpec(memory_space=pltpu.VMEM)
    smem = pl.BlockSpec(memory_space=pltpu.SMEM)
    return pl.pallas_call(
        _gen_body,
        out_shape=jax.ShapeDtypeStruct((_N, _D), f32),
        in_specs=[vmem] * 9 + [smem] * 2,
        out_specs=pl.BlockSpec(memory_space=pltpu.VMEM),
    )(
        z.astype(f32),
        W1.astype(f32),
        W2.reshape(1, _D).astype(f32),
        b2.reshape(1, _D).astype(f32),
        jnp.zeros((8, 512), f32).at[0:1, :].set(We.astype(f32)),
        gat_W.astype(f32),
        gat_b.reshape(3, _D).astype(f32),
        attn_l.reshape(3, _D).astype(f32),
        attn_r.reshape(3, _D).astype(f32),
        b1.reshape(1, 1).astype(f32),
        be.reshape(1, 1).astype(f32),
    )
